# Initial kernel scaffold; baseline (speedup 1.0000x reference)
import jax
import jax.numpy as jnp
from jax.experimental import pallas as pl


def kernel(features, W1, b1, W2, b2, gamma, beta, indices):
    raise NotImplementedError("write your pallas kernel here")


# speedup vs baseline: 5.7016x; 5.7016x over previous
import functools

import jax
import jax.numpy as jnp
from jax import lax
from jax.experimental import pallas as pl
from jax.experimental.pallas import tpu as pltpu
from jax.experimental.pallas import tpu_sc as plsc

L1, N, D = 8192, 2, 512
L2, K = 2048, 16
D_OUT = 512
ND = N * D

NC, NS = 2, 16
NW = NC * NS
ROWS_PER_W = L2 // NW
LANES = 16


def _gather_max_body(feat_hbm, idx_hbm, out_hbm, idx_v, gbuf, out_v, sem0, sem1):
    wid = lax.axis_index("s") * NC + lax.axis_index("c")
    base = wid * ROWS_PER_W
    pltpu.sync_copy(idx_hbm.at[pl.ds(base, ROWS_PER_W)], idx_v)

    sems = (sem0, sem1)

    def start(r, b):
        pltpu.make_async_copy(
            feat_hbm.at[idx_v.at[r]], gbuf.at[b], sems[b]
        ).start()

    def wait(r, b):
        pltpu.make_async_copy(
            feat_hbm.at[idx_v.at[r]], gbuf.at[b], sems[b]
        ).wait()

    start(0, 0)
    start(1, 1)

    def compute(r, b):
        def col_body(c, carry):
            col = c * LANES
            m = gbuf[b, 0, pl.ds(col, LANES)]
            for k in range(1, K):
                m = jnp.maximum(m, gbuf[b, k, pl.ds(col, LANES)])
            out_v[r, pl.ds(col, LANES)] = m
            return carry

        lax.fori_loop(0, ND // LANES, col_body, 0, unroll=2)

    def outer(r0, carry):
        for b in range(2):
            r = r0 + b
            wait(r, b)
            compute(r, b)

            @pl.when(r + 2 < ROWS_PER_W)
            def _():
                start(r + 2, b)

        return carry

    lax.fori_loop(0, ROWS_PER_W // 2, lambda i, c: outer(i * 2, c), 0)

    pltpu.sync_copy(out_v, out_hbm.at[pl.ds(base, ROWS_PER_W)])


def _gather_max(features_flat, indices):
    mesh = plsc.VectorSubcoreMesh(core_axis_name="c", subcore_axis_name="s")
    f = functools.partial(
        pl.kernel,
        out_type=jax.ShapeDtypeStruct((L2, ND), jnp.float32),
        mesh=mesh,
        scratch_types=[
            pltpu.VMEM((ROWS_PER_W, K), jnp.int32),
            pltpu.VMEM((2, K, ND), jnp.float32),
            pltpu.VMEM((ROWS_PER_W, ND), jnp.float32),
            pltpu.SemaphoreType.DMA,
            pltpu.SemaphoreType.DMA,
        ],
    )(_gather_max_body)
    return f(features_flat, indices)


def _mlp_ln_body(x_ref, w1_ref, b1_ref, w2_ref, b2_ref, g_ref, beta_ref, o_ref):
    h = jnp.dot(x_ref[...], w1_ref[...], preferred_element_type=jnp.float32)
    h = jnp.maximum(h + b1_ref[...], 0.0)
    y = jnp.dot(h, w2_ref[...], preferred_element_type=jnp.float32)
    y = y + b2_ref[...]
    mu = jnp.mean(y, axis=-1, keepdims=True)
    var = jnp.mean(jnp.square(y - mu), axis=-1, keepdims=True)
    o_ref[...] = (y - mu) * lax.rsqrt(var + 1e-5) * g_ref[...] + beta_ref[...]


def _mlp_ln(pooled, W1, b1, W2, b2, gamma, beta):
    rows = L2 * N
    tile = 512
    grid = (rows // tile,)
    full = lambda i: (0, 0)
    return pl.pallas_call(
        _mlp_ln_body,
        grid=grid,
        in_specs=[
            pl.BlockSpec((tile, D), lambda i: (i, 0)),
            pl.BlockSpec((D, D_OUT), full),
            pl.BlockSpec((1, D_OUT), full),
            pl.BlockSpec((D_OUT, D_OUT), full),
            pl.BlockSpec((1, D_OUT), full),
            pl.BlockSpec((1, D_OUT), full),
            pl.BlockSpec((1, D_OUT), full),
        ],
        out_specs=pl.BlockSpec((tile, D_OUT), lambda i: (i, 0)),
        out_shape=jax.ShapeDtypeStruct((rows, D_OUT), jnp.float32),
    )(pooled, W1, b1, W2, b2, gamma, beta)


def kernel(features, W1, b1, W2, b2, gamma, beta, indices):
    features_flat = features.reshape(L1, ND)
    idx = indices.astype(jnp.int32)
    pooled = _gather_max(features_flat, idx)
    x = pooled.reshape(L2 * N, D)
    out = _mlp_ln(
        x,
        W1,
        b1.reshape(1, D_OUT),
        W2,
        b2.reshape(1, D_OUT),
        gamma.reshape(1, D_OUT),
        beta.reshape(1, D_OUT),
    )
    return out.reshape(L2, N, D_OUT)


# speedup vs baseline: 5.8470x; 1.0255x over previous
import functools

import jax
import jax.numpy as jnp
from jax import lax
from jax.experimental import pallas as pl
from jax.experimental.pallas import tpu as pltpu
from jax.experimental.pallas import tpu_sc as plsc

L1, N, D = 8192, 2, 512
L2, K = 2048, 16
D_OUT = 512
ND = N * D

NC, NS = 2, 16
NW = NC * NS
ROWS_PER_W = L2 // NW
LANES = 16


def _gather_max_body(feat_hbm, idx_hbm, out_hbm, idx_v, gbuf, out_v, sem0, sem1):
    wid = lax.axis_index("s") * NC + lax.axis_index("c")
    base = wid * ROWS_PER_W
    pltpu.sync_copy(idx_hbm.at[pl.ds(base, ROWS_PER_W)], idx_v)

    sems = (sem0, sem1)

    def start(r, b):
        pltpu.make_async_copy(
            feat_hbm.at[idx_v.at[r]], gbuf.at[b], sems[b]
        ).start()

    def wait(r, b):
        pltpu.make_async_copy(
            feat_hbm.at[idx_v.at[r]], gbuf.at[b], sems[b]
        ).wait()

    start(0, 0)
    start(1, 1)

    def compute(r, b):
        def col_body(c, carry):
            col = c * LANES
            for n in range(N):
                m = gbuf[b, 0, n, pl.ds(col, LANES)]
                for k in range(1, K):
                    m = jnp.maximum(m, gbuf[b, k, n, pl.ds(col, LANES)])
                out_v[r, n, pl.ds(col, LANES)] = m
            return carry

        lax.fori_loop(0, D // LANES, col_body, 0, unroll=2)

    def outer(r0, carry):
        for b in range(2):
            r = r0 + b
            wait(r, b)
            compute(r, b)

            @pl.when(r + 2 < ROWS_PER_W)
            def _():
                start(r + 2, b)

        return carry

    lax.fori_loop(0, ROWS_PER_W // 2, lambda i, c: outer(i * 2, c), 0)

    pltpu.sync_copy(out_v, out_hbm.at[pl.ds(base, ROWS_PER_W)])


def _gather_max(features, indices):
    mesh = plsc.VectorSubcoreMesh(core_axis_name="c", subcore_axis_name="s")
    f = functools.partial(
        pl.kernel,
        out_type=jax.ShapeDtypeStruct((L2, N, D), jnp.float32),
        mesh=mesh,
        scratch_types=[
            pltpu.VMEM((ROWS_PER_W, K), jnp.int32),
            pltpu.VMEM((2, K, N, D), jnp.float32),
            pltpu.VMEM((ROWS_PER_W, N, D), jnp.float32),
            pltpu.SemaphoreType.DMA,
            pltpu.SemaphoreType.DMA,
        ],
    )(_gather_max_body)
    return f(features, indices)


def _mlp_ln_body(x_ref, w1_ref, b1_ref, w2_ref, b2_ref, g_ref, beta_ref, o_ref):
    h = jnp.dot(x_ref[...], w1_ref[...], preferred_element_type=jnp.float32)
    h = jnp.maximum(h + b1_ref[...], 0.0)
    y = jnp.dot(h, w2_ref[...], preferred_element_type=jnp.float32)
    y = y + b2_ref[...]
    mu = jnp.mean(y, axis=-1, keepdims=True)
    var = jnp.mean(jnp.square(y - mu), axis=-1, keepdims=True)
    o_ref[...] = (y - mu) * lax.rsqrt(var + 1e-5) * g_ref[...] + beta_ref[...]


def _mlp_ln(pooled, W1, b1, W2, b2, gamma, beta):
    rows = L2 * N
    tile = 512
    grid = (rows // tile,)
    full = lambda i: (0, 0)
    return pl.pallas_call(
        _mlp_ln_body,
        grid=grid,
        in_specs=[
            pl.BlockSpec((tile, D), lambda i: (i, 0)),
            pl.BlockSpec((D, D_OUT), full),
            pl.BlockSpec((1, D_OUT), full),
            pl.BlockSpec((D_OUT, D_OUT), full),
            pl.BlockSpec((1, D_OUT), full),
            pl.BlockSpec((1, D_OUT), full),
            pl.BlockSpec((1, D_OUT), full),
        ],
        out_specs=pl.BlockSpec((tile, D_OUT), lambda i: (i, 0)),
        out_shape=jax.ShapeDtypeStruct((rows, D_OUT), jnp.float32),
    )(pooled, W1, b1, W2, b2, gamma, beta)


def kernel(features, W1, b1, W2, b2, gamma, beta, indices):
    idx = indices.astype(jnp.int32)
    pooled = _gather_max(features, idx)
    x = pooled.reshape(L2 * N, D)
    out = _mlp_ln(
        x,
        W1,
        b1.reshape(1, D_OUT),
        W2,
        b2.reshape(1, D_OUT),
        gamma.reshape(1, D_OUT),
        beta.reshape(1, D_OUT),
    )
    return out.reshape(L2, N, D_OUT)


# speedup vs baseline: 6.4457x; 1.1024x over previous
import functools

import jax
import jax.numpy as jnp
from jax import lax
from jax.experimental import pallas as pl
from jax.experimental.pallas import tpu as pltpu
from jax.experimental.pallas import tpu_sc as plsc

L1, N, D = 8192, 2, 512
L2, K = 2048, 16
D_OUT = 512
DH = D // 2

NC, NS = 2, 16
NW = NC * NS
ROWS_PER_W = L2 // NW
LANES = 16


def _pack_body(x_ref, o_ref):
    x = x_ref[...]
    lo = x[..., :DH].astype(jnp.bfloat16)
    hi = x[..., DH:].astype(jnp.bfloat16)
    lo32 = lax.convert_element_type(
        lax.bitcast_convert_type(lo, jnp.uint16), jnp.uint32)
    hi32 = lax.convert_element_type(
        lax.bitcast_convert_type(hi, jnp.uint16), jnp.uint32)
    o_ref[...] = lax.bitcast_convert_type(lo32 | (hi32 << 16), jnp.int32)


def _pack_bf16(features):
    tile = 1024
    return pl.pallas_call(
        _pack_body,
        grid=(L1 // tile,),
        in_specs=[pl.BlockSpec((tile, N, D), lambda i: (i, 0, 0))],
        out_specs=pl.BlockSpec((tile, N, DH), lambda i: (i, 0, 0)),
        out_shape=jax.ShapeDtypeStruct((L1, N, DH), jnp.int32),
    )(features)


def _gather_max_body(feat_hbm, idx_hbm, out_hbm, idx_v, gbuf, out_v, sem0, sem1):
    wid = lax.axis_index("s") * NC + lax.axis_index("c")
    base = wid * ROWS_PER_W
    pltpu.sync_copy(idx_hbm.at[pl.ds(base, ROWS_PER_W)], idx_v)

    sems = (sem0, sem1)

    def start(r, b):
        pltpu.make_async_copy(
            feat_hbm.at[idx_v.at[r]], gbuf.at[b], sems[b]
        ).start()

    def wait(r, b):
        pltpu.make_async_copy(
            feat_hbm.at[idx_v.at[r]], gbuf.at[b], sems[b]
        ).wait()

    start(0, 0)
    start(1, 1)

    gb = gbuf.bitcast(jnp.bfloat16)
    ob = out_v.bitcast(jnp.bfloat16)

    def compute(r, b):
        def col_body(c, carry):
            col = c * (2 * LANES)
            for n in range(N):
                m = gb[b, 0, n, pl.ds(col, 2 * LANES)]
                for k in range(1, K):
                    m = jnp.maximum(m, gb[b, k, n, pl.ds(col, 2 * LANES)])
                ob[r, n, pl.ds(col, 2 * LANES)] = m
            return carry

        lax.fori_loop(0, DH // LANES, col_body, 0, unroll=2)

    def outer(r0, carry):
        for b in range(2):
            r = r0 + b
            wait(r, b)
            compute(r, b)

            @pl.when(r + 2 < ROWS_PER_W)
            def _():
                start(r + 2, b)

        return carry

    lax.fori_loop(0, ROWS_PER_W // 2, lambda i, c: outer(i * 2, c), 0)

    pltpu.sync_copy(out_v, out_hbm.at[pl.ds(base, ROWS_PER_W)])


def _gather_max(feat_packed, indices):
    mesh = plsc.VectorSubcoreMesh(core_axis_name="c", subcore_axis_name="s")
    f = functools.partial(
        pl.kernel,
        out_type=jax.ShapeDtypeStruct((L2, N, DH), jnp.int32),
        mesh=mesh,
        scratch_types=[
            pltpu.VMEM((ROWS_PER_W, K), jnp.int32),
            pltpu.VMEM((2, K, N, DH), jnp.int32),
            pltpu.VMEM((ROWS_PER_W, N, DH), jnp.int32),
            pltpu.SemaphoreType.DMA,
            pltpu.SemaphoreType.DMA,
        ],
    )(_gather_max_body)
    return f(feat_packed, indices)


def _mlp_ln_body(x_ref, w1_ref, b1_ref, w2_ref, b2_ref, g_ref, beta_ref, o_ref):
    xi = lax.bitcast_convert_type(x_ref[...], jnp.uint32)
    lo = lax.bitcast_convert_type(
        lax.convert_element_type(xi & 0xFFFF, jnp.uint16), jnp.bfloat16)
    hi = lax.bitcast_convert_type(
        lax.convert_element_type(xi >> 16, jnp.uint16), jnp.bfloat16)
    x = jnp.concatenate([lo, hi], axis=-1)
    h = jnp.dot(x, w1_ref[...], preferred_element_type=jnp.float32)
    h = jnp.maximum(h + b1_ref[...], 0.0)
    y = jnp.dot(h.astype(jnp.bfloat16), w2_ref[...],
                preferred_element_type=jnp.float32)
    y = y + b2_ref[...]
    mu = jnp.mean(y, axis=-1, keepdims=True)
    var = jnp.mean(jnp.square(y - mu), axis=-1, keepdims=True)
    o_ref[...] = (y - mu) * lax.rsqrt(var + 1e-5) * g_ref[...] + beta_ref[...]


def _mlp_ln(packed, W1, b1, W2, b2, gamma, beta):
    rows = L2 * N
    tile = 512
    grid = (rows // tile,)
    full = lambda i: (0, 0)
    return pl.pallas_call(
        _mlp_ln_body,
        grid=grid,
        in_specs=[
            pl.BlockSpec((tile, DH), lambda i: (i, 0)),
            pl.BlockSpec((D, D_OUT), full),
            pl.BlockSpec((1, D_OUT), full),
            pl.BlockSpec((D_OUT, D_OUT), full),
            pl.BlockSpec((1, D_OUT), full),
            pl.BlockSpec((1, D_OUT), full),
            pl.BlockSpec((1, D_OUT), full),
        ],
        out_specs=pl.BlockSpec((tile, D_OUT), lambda i: (i, 0)),
        out_shape=jax.ShapeDtypeStruct((rows, D_OUT), jnp.float32),
    )(packed, W1, b1, W2, b2, gamma, beta)


def kernel(features, W1, b1, W2, b2, gamma, beta, indices):
    idx = indices.astype(jnp.int32)
    feat_packed = _pack_bf16(features)
    pooled = _gather_max(feat_packed, idx)
    x = pooled.reshape(L2 * N, DH)
    out = _mlp_ln(
        x,
        W1.astype(jnp.bfloat16),
        b1.reshape(1, D_OUT),
        W2.astype(jnp.bfloat16),
        b2.reshape(1, D_OUT),
        gamma.reshape(1, D_OUT),
        beta.reshape(1, D_OUT),
    )
    return out.reshape(L2, N, D_OUT)
